# baseline (speedup 1.0000x reference)
import functools

import jax
import jax.numpy as jnp
from jax import lax
from jax.experimental import pallas as pl
from jax.experimental.pallas import tpu as pltpu
from jax.experimental.pallas import tpu_sc as plsc

_L = 16
_C = 64


def _bprmf_kernel(B, N, D, n_workers):
    bpw = B // n_workers
    n_chunks = bpw // _C
    mesh = plsc.VectorSubcoreMesh(core_axis_name="c", subcore_axis_name="s")

    @functools.partial(
        pl.kernel,
        mesh=mesh,
        compiler_params=pltpu.CompilerParams(needs_layout_passes=False),
        out_type=jax.ShapeDtypeStruct((B,), jnp.float32),
        scratch_types=[
            pltpu.VMEM((bpw,), jnp.int32),
            pltpu.VMEM((bpw,), jnp.int32),
            pltpu.VMEM((_C * 64,), jnp.int32),
            pltpu.VMEM((_C * 64,), jnp.int32),
            pltpu.VMEM((_C * 64,), jnp.float32),
            pltpu.VMEM((_C * 64,), jnp.float32),
            pltpu.VMEM((bpw,), jnp.float32),
            pltpu.VMEM((bpw,), jnp.float32),
            pltpu.SemaphoreType.DMA,
            pltpu.SemaphoreType.DMA,
            pltpu.SemaphoreType.DMA,
        ],
    )
    def run(u_hbm, i_hbm, p_hbm, q_hbm, b_hbm, out_hbm,
            uv, iv, pidx, qidx, pv, qv, bv, ov, semb, semp, semq):
        wid = lax.axis_index("s") * 2 + lax.axis_index("c")
        base = wid * bpw
        pltpu.sync_copy(u_hbm.at[pl.ds(base, bpw)], uv)
        pltpu.sync_copy(i_hbm.at[pl.ds(base, bpw)], iv)
        cp_b = pltpu.async_copy(b_hbm.at[iv], bv, semb)

        def chunk_body(ch, _):
            c0 = ch * _C
            for g in range(_C // _L):
                usl = uv[pl.ds(c0 + g * _L, _L)]
                isl = iv[pl.ds(c0 + g * _L, _L)]
                for c in range(D):
                    pidx[pl.ds(c * _C + g * _L, _L)] = usl + (c * N)
                    qidx[pl.ds(c * _C + g * _L, _L)] = isl + (c * N)
            cp_p = pltpu.async_copy(p_hbm.at[pidx], pv, semp)
            cp_q = pltpu.async_copy(q_hbm.at[qidx], qv, semq)
            cp_p.wait()
            cp_q.wait()
            for g in range(_C // _L):
                acc = jnp.zeros((_L,), jnp.float32)
                for c in range(D):
                    sl = pl.ds(c * _C + g * _L, _L)
                    acc = acc + pv[sl] * qv[sl]
                ov[pl.ds(c0 + g * _L, _L)] = acc
            return 0

        lax.fori_loop(0, n_chunks, chunk_body, 0)
        cp_b.wait()

        def bias_body(g, _):
            sl = pl.ds(g * _L, _L)
            ov[sl] = ov[sl] + bv[sl]
            return 0

        lax.fori_loop(0, bpw // _L, bias_body, 0)
        pltpu.sync_copy(ov, out_hbm.at[pl.ds(base, bpw)])

    return run


def kernel(u, i, P, Q, bi):
    B = u.shape[0]
    N, D = P.shape
    Pf = P.T.ravel()
    Qf = Q.T.ravel()
    return _bprmf_kernel(B, N, D, 32)(u, i, Pf, Qf, bi.reshape(-1))


# speedup vs baseline: 9.0833x; 9.0833x over previous
import functools

import jax
import jax.numpy as jnp
from jax import lax
from jax.experimental import pallas as pl
from jax.experimental.pallas import tpu as pltpu
from jax.experimental.pallas import tpu_sc as plsc

_L = 16
_C = 32


def _bprmf_kernel(B, N, D, n_workers):
    bpw = B // n_workers
    n_chunks = bpw // _C
    mesh = plsc.VectorSubcoreMesh(core_axis_name="c", subcore_axis_name="s")

    @functools.partial(
        pl.kernel,
        mesh=mesh,
        compiler_params=pltpu.CompilerParams(needs_layout_passes=False),
        out_type=jax.ShapeDtypeStruct((B,), jnp.float32),
        scratch_types=[
            pltpu.VMEM((bpw,), jnp.int32),
            pltpu.VMEM((bpw,), jnp.int32),
            pltpu.VMEM((_C,), jnp.int32),
            pltpu.VMEM((_C,), jnp.int32),
            pltpu.VMEM((_C, 8, 128), jnp.float32),
            pltpu.VMEM((_C, 8, 128), jnp.float32),
            pltpu.VMEM((bpw,), jnp.float32),
            pltpu.VMEM((bpw,), jnp.float32),
            pltpu.SemaphoreType.DMA,
            pltpu.SemaphoreType.DMA,
            pltpu.SemaphoreType.DMA,
        ],
    )
    def run(u_hbm, i_hbm, p_hbm, q_hbm, b_hbm, out_hbm,
            uv, iv, pidx, qidx, pv, qv, bv, ov, semb, semp, semq):
        wid = lax.axis_index("s") * 2 + lax.axis_index("c")
        base = wid * bpw
        pltpu.sync_copy(u_hbm.at[pl.ds(base, bpw)], uv)
        pltpu.sync_copy(i_hbm.at[pl.ds(base, bpw)], iv)
        cp_b = pltpu.async_copy(b_hbm.at[iv], bv, semb)

        lanes = lax.iota(jnp.int32, _L)

        def chunk_body(ch, _):
            c0 = ch * _C
            for g in range(_C // _L):
                usl = uv[pl.ds(c0 + g * _L, _L)]
                isl = iv[pl.ds(c0 + g * _L, _L)]
                pidx[pl.ds(g * _L, _L)] = usl >> 3
                qidx[pl.ds(g * _L, _L)] = isl >> 3
            cp_p = pltpu.async_copy(p_hbm.at[pidx], pv, semp)
            cp_q = pltpu.async_copy(q_hbm.at[qidx], qv, semq)
            cp_p.wait()
            cp_q.wait()
            for g in range(_C // _L):
                usl = uv[pl.ds(c0 + g * _L, _L)]
                isl = iv[pl.ds(c0 + g * _L, _L)]
                kvec = g * _L + lanes
                psub = usl & 7
                qsub = isl & 7
                acc = jnp.zeros((_L,), jnp.float32)
                for c in range(D):
                    cvec = jnp.full((_L,), c, jnp.int32)
                    pcol = plsc.load_gather(pv, [kvec, psub, cvec])
                    qcol = plsc.load_gather(qv, [kvec, qsub, cvec])
                    acc = acc + pcol * qcol
                ov[pl.ds(c0 + g * _L, _L)] = acc
            return 0

        lax.fori_loop(0, n_chunks, chunk_body, 0)
        cp_b.wait()

        def bias_body(g, _):
            sl = pl.ds(g * _L, _L)
            ov[sl] = ov[sl] + bv[sl]
            return 0

        lax.fori_loop(0, bpw // _L, bias_body, 0)
        pltpu.sync_copy(ov, out_hbm.at[pl.ds(base, bpw)])

    return run


def kernel(u, i, P, Q, bi):
    B = u.shape[0]
    N, D = P.shape
    z = jnp.zeros((N // 8, 8, 128 - D), jnp.float32)
    P3 = jnp.concatenate([P.reshape(N // 8, 8, D), z], axis=2)
    Q3 = jnp.concatenate([Q.reshape(N // 8, 8, D), z], axis=2)
    return _bprmf_kernel(B, N, D, 32)(u, i, P3, Q3, bi.reshape(-1))


# speedup vs baseline: 23.1981x; 2.5539x over previous
import functools

import jax
import jax.numpy as jnp
from jax import lax
from jax.experimental import pallas as pl
from jax.experimental.pallas import tpu as pltpu
from jax.experimental.pallas import tpu_sc as plsc

_L = 16
_R = 4


def _bprmf_kernel(B, N, D, n_workers):
    bpw = B // n_workers
    n_groups = bpw // _L
    mesh = plsc.VectorSubcoreMesh(core_axis_name="c", subcore_axis_name="s")

    @functools.partial(
        pl.kernel,
        mesh=mesh,
        compiler_params=pltpu.CompilerParams(
            needs_layout_passes=False, use_tc_tiling_on_sc=True),
        out_type=jax.ShapeDtypeStruct((B,), jnp.float32),
        scratch_types=[
            pltpu.VMEM((bpw,), jnp.int32),
            pltpu.VMEM((bpw,), jnp.int32),
            pltpu.VMEM((_R, D, 128), jnp.float32),
            pltpu.VMEM((_R, D, 128), jnp.float32),
            pltpu.VMEM((bpw,), jnp.float32),
            pltpu.VMEM((bpw,), jnp.float32),
            pltpu.SemaphoreType.DMA,
        ] + [pltpu.SemaphoreType.DMA] * (2 * _R),
    )
    def run(u_hbm, i_hbm, pt_hbm, qt_hbm, b_hbm, out_hbm,
            uv, iv, pv, qv, bv, ov, semb, *sems):
        psems, qsems = sems[:_R], sems[_R:]
        wid = lax.axis_index("s") * 2 + lax.axis_index("c")
        base = wid * bpw
        pltpu.sync_copy(u_hbm.at[pl.ds(base, bpw)], uv)
        pltpu.sync_copy(i_hbm.at[pl.ds(base, bpw)], iv)
        cp_b = pltpu.async_copy(b_hbm.at[iv], bv, semb)

        lanes = lax.iota(jnp.int32, _L)

        def issue(ublkv, iblkv, j, s):
            pltpu.async_copy(
                pt_hbm.at[:, pl.ds(ublkv[j] * 128, 128)], pv.at[s], psems[s])
            pltpu.async_copy(
                qt_hbm.at[:, pl.ds(iblkv[j] * 128, 128)], qv.at[s], qsems[s])

        def wait_slot(s):
            pltpu.make_async_copy(
                pt_hbm.at[:, pl.ds(0, 128)], pv.at[s], psems[s]).wait()
            pltpu.make_async_copy(
                qt_hbm.at[:, pl.ds(0, 128)], qv.at[s], qsems[s]).wait()

        def dot_at(s, cu, ci):
            svec = jnp.full((_L,), s, jnp.int32)
            cuv = jnp.full((_L,), cu, jnp.int32)
            civ = jnp.full((_L,), ci, jnp.int32)
            acc = jnp.zeros((_L,), jnp.float32)
            for f in range(D // _L):
                fvec = f * _L + lanes
                pcol = plsc.load_gather(pv, [svec, fvec, cuv])
                qcol = plsc.load_gather(qv, [svec, fvec, civ])
                acc = acc + pcol * qcol
            return jnp.sum(acc)

        def group_body(g, carry):
            res_prev, pucolv, picolv = carry
            uvec = uv[pl.ds(g * _L, _L)]
            ivec = iv[pl.ds(g * _L, _L)]
            ublkv = uvec >> 7
            iblkv = ivec >> 7
            ucolv = uvec & 127
            icolv = ivec & 127
            res = jnp.zeros((_L,), jnp.float32)
            for j in range(_L):
                s = j % _R
                if j < _R:
                    jl = _L - _R + j

                    @pl.when(g > 0)
                    def _():
                        wait_slot(s)

                    d = dot_at(s, pucolv[jl], picolv[jl])
                    res_prev = jnp.where(lanes == jl, d, res_prev)
                    if j == _R - 1:
                        @pl.when(g > 0)
                        def _():
                            ov[pl.ds((g - 1) * _L, _L)] = res_prev
                else:
                    jl = j - _R
                    wait_slot(s)
                    d = dot_at(s, ucolv[jl], icolv[jl])
                    res = jnp.where(lanes == jl, d, res)
                issue(ublkv, iblkv, j, s)
            return res, ucolv, icolv

        init = (jnp.zeros((_L,), jnp.float32),
                jnp.zeros((_L,), jnp.int32), jnp.zeros((_L,), jnp.int32))
        res, ucolv, icolv = lax.fori_loop(0, n_groups, group_body, init)

        for t in range(_R):
            jl = _L - _R + t
            wait_slot(t)
            d = dot_at(t, ucolv[jl], icolv[jl])
            res = jnp.where(lanes == jl, d, res)
        ov[pl.ds((n_groups - 1) * _L, _L)] = res

        cp_b.wait()

        def bias_body(g, _):
            sl = pl.ds(g * _L, _L)
            ov[sl] = ov[sl] + bv[sl]
            return 0

        lax.fori_loop(0, bpw // _L, bias_body, 0)
        pltpu.sync_copy(ov, out_hbm.at[pl.ds(base, bpw)])

    return run


def kernel(u, i, P, Q, bi):
    B = u.shape[0]
    N, D = P.shape
    return _bprmf_kernel(B, N, D, 32)(
        u, i, jnp.transpose(P), jnp.transpose(Q),
        jnp.transpose(bi).reshape(-1))


# speedup vs baseline: 25.1610x; 1.0846x over previous
import functools

import jax
import jax.numpy as jnp
from jax import lax
from jax.experimental import pallas as pl
from jax.experimental.pallas import tpu as pltpu
from jax.experimental.pallas import tpu_sc as plsc

_L = 16
_R = 4


def _bprmf_kernel(B, N, D, n_workers):
    bpw = B // n_workers
    n_groups = bpw // _L
    mesh = plsc.VectorSubcoreMesh(core_axis_name="c", subcore_axis_name="s")

    @functools.partial(
        pl.kernel,
        mesh=mesh,
        compiler_params=pltpu.CompilerParams(
            needs_layout_passes=False, use_tc_tiling_on_sc=True),
        out_type=jax.ShapeDtypeStruct((B,), jnp.float32),
        scratch_types=[
            pltpu.VMEM((bpw,), jnp.int32),
            pltpu.VMEM((bpw,), jnp.int32),
            pltpu.VMEM((_R, D, 128), jnp.float32),
            pltpu.VMEM((_R, D, 128), jnp.float32),
            pltpu.VMEM((bpw,), jnp.float32),
        ] + [pltpu.SemaphoreType.DMA] * (2 * _R),
    )
    def run(u_hbm, i_hbm, pt_hbm, qt_hbm, out_hbm,
            uv, iv, pv, qv, ov, *sems):
        psems, qsems = sems[:_R], sems[_R:]
        wid = lax.axis_index("s") * 2 + lax.axis_index("c")
        base = wid * bpw
        pltpu.sync_copy(u_hbm.at[pl.ds(base, bpw)], uv)
        pltpu.sync_copy(i_hbm.at[pl.ds(base, bpw)], iv)

        lanes = lax.iota(jnp.int32, _L)

        def issue(ublkv, iblkv, j, s):
            pltpu.async_copy(
                pt_hbm.at[:, pl.ds(ublkv[j] * 128, 128)], pv.at[s], psems[s])
            pltpu.async_copy(
                qt_hbm.at[:, pl.ds(iblkv[j] * 128, 128)], qv.at[s], qsems[s])

        def wait_slot(s):
            pltpu.make_async_copy(
                pt_hbm.at[:, pl.ds(0, 128)], pv.at[s], psems[s]).wait()
            pltpu.make_async_copy(
                qt_hbm.at[:, pl.ds(0, 128)], qv.at[s], qsems[s]).wait()

        def dot_at(s, cu, ci):
            svec = jnp.full((_L,), s, jnp.int32)
            cuv = jnp.full((_L,), cu, jnp.int32)
            civ = jnp.full((_L,), ci, jnp.int32)
            acc = jnp.zeros((_L,), jnp.float32)
            for f in range(D // _L):
                fvec = f * _L + lanes
                pcol = plsc.load_gather(pv, [svec, fvec, cuv])
                qcol = plsc.load_gather(qv, [svec, fvec, civ])
                acc = acc + pcol * qcol
            return jnp.sum(acc)

        def group_body(g, carry):
            res_prev, pucolv, picolv = carry
            uvec = uv[pl.ds(g * _L, _L)]
            ivec = iv[pl.ds(g * _L, _L)]
            ublkv = uvec >> 7
            iblkv = ivec >> 7
            ucolv = uvec & 127
            icolv = ivec & 127
            res = jnp.zeros((_L,), jnp.float32)
            for j in range(_L):
                s = j % _R
                if j < _R:
                    jl = _L - _R + j

                    @pl.when(g > 0)
                    def _():
                        wait_slot(s)

                    d = dot_at(s, pucolv[jl], picolv[jl])
                    res_prev = jnp.where(lanes == jl, d, res_prev)
                    if j == _R - 1:
                        @pl.when(g > 0)
                        def _():
                            ov[pl.ds((g - 1) * _L, _L)] = res_prev
                else:
                    jl = j - _R
                    wait_slot(s)
                    d = dot_at(s, ucolv[jl], icolv[jl])
                    res = jnp.where(lanes == jl, d, res)
                issue(ublkv, iblkv, j, s)
            return res, ucolv, icolv

        init = (jnp.zeros((_L,), jnp.float32),
                jnp.zeros((_L,), jnp.int32), jnp.zeros((_L,), jnp.int32))
        res, ucolv, icolv = lax.fori_loop(0, n_groups, group_body, init)

        for t in range(_R):
            jl = _L - _R + t
            wait_slot(t)
            d = dot_at(t, ucolv[jl], icolv[jl])
            res = jnp.where(lanes == jl, d, res)
        ov[pl.ds((n_groups - 1) * _L, _L)] = res

        pltpu.sync_copy(ov, out_hbm.at[pl.ds(base, bpw)])

    return run


def _bias_kernel(B, n_workers):
    bpw = B // n_workers
    mesh = plsc.VectorSubcoreMesh(core_axis_name="c", subcore_axis_name="s")

    @functools.partial(
        pl.kernel,
        mesh=mesh,
        compiler_params=pltpu.CompilerParams(
            needs_layout_passes=False, use_tc_tiling_on_sc=True),
        out_type=jax.ShapeDtypeStruct((B,), jnp.float32),
        scratch_types=[
            pltpu.VMEM((bpw,), jnp.int32),
            pltpu.VMEM((bpw,), jnp.float32),
            pltpu.VMEM((bpw,), jnp.float32),
            pltpu.SemaphoreType.DMA,
        ],
    )
    def run(d_hbm, i_hbm, b_hbm, out_hbm, iv, bv, dv, semb):
        wid = lax.axis_index("s") * 2 + lax.axis_index("c")
        base = wid * bpw
        pltpu.sync_copy(i_hbm.at[pl.ds(base, bpw)], iv)
        cp_b = pltpu.async_copy(b_hbm.at[iv], bv, semb)
        pltpu.sync_copy(d_hbm.at[pl.ds(base, bpw)], dv)
        cp_b.wait()

        def body(g, _):
            sl = pl.ds(g * _L, _L)
            dv[sl] = dv[sl] + bv[sl]
            return 0

        lax.fori_loop(0, bpw // _L, body, 0)
        pltpu.sync_copy(dv, out_hbm.at[pl.ds(base, bpw)])

    return run


def kernel(u, i, P, Q, bi):
    B = u.shape[0]
    N, D = P.shape
    dots = _bprmf_kernel(B, N, D, 32)(
        u, i, jnp.transpose(P), jnp.transpose(Q))
    return _bias_kernel(B, 32)(dots, i, jnp.transpose(bi).reshape(-1))
